# Initial kernel scaffold; baseline (speedup 1.0000x reference)
import jax
import jax.numpy as jnp
from jax.experimental import pallas as pl


def kernel(input, embed):
    raise NotImplementedError("write your pallas kernel here")


# speedup vs baseline: 1.3232x; 1.3232x over previous
import functools

import jax
import jax.numpy as jnp
from jax import lax
from jax.experimental import pallas as pl
from jax.experimental.pallas import tpu as pltpu
from jax.experimental.pallas import tpu_sc as plsc

_DIM = 256
_NE = 8192
_NTOK = 8192
_M_BLK = 256


def _dist_argmin_body(x_ref, e_ref, ind_ref, cn_ref):
    @pl.when(pl.program_id(0) == 0)
    def _():
        e = e_ref[...]
        cn_ref[...] = jnp.sum(e * e, axis=0, keepdims=True)

    x = x_ref[...]
    rn = jnp.sum(x * x, axis=1, keepdims=True)
    mm = lax.dot_general(
        x, e_ref[...],
        dimension_numbers=(((1,), (0,)), ((), ())),
        preferred_element_type=jnp.float32,
    )
    d = rn - 2.0 * mm + cn_ref[...]
    m = jnp.min(d, axis=1, keepdims=True)
    iota = lax.broadcasted_iota(jnp.int32, d.shape, 1)
    ind = jnp.min(jnp.where(d == m, iota, jnp.int32(2**31 - 1)), axis=1)
    ind_ref[...] = ind


def _dist_argmin(flatten, embed):
    grid = (_NTOK // _M_BLK,)
    return pl.pallas_call(
        _dist_argmin_body,
        grid=grid,
        in_specs=[
            pl.BlockSpec((_M_BLK, _DIM), lambda i: (i, 0)),
            pl.BlockSpec((_DIM, _NE), lambda i: (0, 0)),
        ],
        out_specs=pl.BlockSpec((_M_BLK,), lambda i: (i,)),
        out_shape=jax.ShapeDtypeStruct((_NTOK,), jnp.int32),
        scratch_shapes=[pltpu.VMEM((1, _NE), jnp.float32)],
    )(flatten, embed)


_NW = 32
_B_PER_W = _NTOK // _NW
_CHUNK = 128


def _sc_gather_body(table_hbm, idx_hbm, x_hbm, qst_hbm, loss_hbm,
                    idx_v, rows_v, x_v, acc_v, sem):
    wid = lax.axis_index("s") * 2 + lax.axis_index("c")
    base = wid * _B_PER_W
    acc = jnp.zeros((16,), jnp.float32)
    for h in range(_B_PER_W // _CHUNK):
        rbase = base + h * _CHUNK
        pltpu.sync_copy(idx_hbm.at[pl.ds(rbase, _CHUNK)], idx_v)
        pltpu.async_copy(table_hbm.at[idx_v], rows_v, sem).wait()
        pltpu.sync_copy(x_hbm.at[pl.ds(rbase, _CHUNK)], x_v)

        def body(r, acc):
            for c in range(_DIM // 16):
                q = rows_v[r, pl.ds(c * 16, 16)]
                xv = x_v[r, pl.ds(c * 16, 16)]
                dv = q - xv
                rows_v[r, pl.ds(c * 16, 16)] = xv + dv
                acc = acc + dv * dv
            return acc

        acc = lax.fori_loop(0, _CHUNK, body, acc)
        pltpu.sync_copy(rows_v, qst_hbm.at[pl.ds(rbase, _CHUNK)])
    acc_v[...] = acc
    pltpu.sync_copy(acc_v, loss_hbm.at[wid])


def _sc_gather(embed_t, ind_flat, flatten):
    mesh = plsc.VectorSubcoreMesh(core_axis_name="c", subcore_axis_name="s")
    fn = functools.partial(
        pl.kernel,
        mesh=mesh,
        out_type=[
            jax.ShapeDtypeStruct((_NTOK, _DIM), jnp.float32),
            jax.ShapeDtypeStruct((_NW, 16), jnp.float32),
        ],
        scratch_types=[
            pltpu.VMEM((_CHUNK,), jnp.int32),
            pltpu.VMEM((_CHUNK, _DIM), jnp.float32),
            pltpu.VMEM((_CHUNK, _DIM), jnp.float32),
            pltpu.VMEM((16,), jnp.float32),
            pltpu.SemaphoreType.DMA,
        ],
    )(_sc_gather_body)
    return fn(embed_t, ind_flat, flatten)


def kernel(input, embed):
    flatten = input.reshape(_NTOK, _DIM)
    ind_flat = _dist_argmin(flatten, embed)
    qst_flat, loss_partials = _sc_gather(embed.T, ind_flat, flatten)
    quantize_st = qst_flat.reshape(input.shape)
    embed_ind = ind_flat.reshape(input.shape[:-1])
    commit_loss = jnp.sum(loss_partials) / jnp.float32(_NTOK * _DIM)
    return quantize_st, embed_ind, commit_loss


# speedup vs baseline: 1.3705x; 1.0357x over previous
import functools

import jax
import jax.numpy as jnp
from jax import lax
from jax.experimental import pallas as pl
from jax.experimental.pallas import tpu as pltpu
from jax.experimental.pallas import tpu_sc as plsc

_DIM = 256
_NE = 8192
_NTOK = 8192
_M_BLK = 256


def _dist_argmin_body(x_ref, e_ref, ind_ref, cn_ref):
    @pl.when(pl.program_id(0) == 0)
    def _():
        e = e_ref[...]
        cn_ref[...] = jnp.sum(e * e, axis=0, keepdims=True)

    x = x_ref[...]
    rn = jnp.sum(x * x, axis=1, keepdims=True)
    mm = lax.dot_general(
        x, e_ref[...],
        dimension_numbers=(((1,), (0,)), ((), ())),
        preferred_element_type=jnp.float32,
    )
    d = rn - 2.0 * mm + cn_ref[...]
    ind_ref[...] = jnp.argmin(d, axis=1).astype(jnp.int32)


def _dist_argmin(flatten, embed):
    grid = (_NTOK // _M_BLK,)
    return pl.pallas_call(
        _dist_argmin_body,
        grid=grid,
        in_specs=[
            pl.BlockSpec((_M_BLK, _DIM), lambda i: (i, 0)),
            pl.BlockSpec((_DIM, _NE), lambda i: (0, 0)),
        ],
        out_specs=pl.BlockSpec((_M_BLK,), lambda i: (i,)),
        out_shape=jax.ShapeDtypeStruct((_NTOK,), jnp.int32),
        scratch_shapes=[pltpu.VMEM((1, _NE), jnp.float32)],
    )(flatten, embed)


_NW = 32
_B_PER_W = _NTOK // _NW
_CHUNK = 128


def _sc_gather_body(table_hbm, idx_hbm, x_hbm, qst_hbm, loss_hbm,
                    idx_v, rows_v, x_v, acc_v, sem):
    wid = lax.axis_index("s") * 2 + lax.axis_index("c")
    base = wid * _B_PER_W
    acc = jnp.zeros((16,), jnp.float32)
    for h in range(_B_PER_W // _CHUNK):
        rbase = base + h * _CHUNK
        pltpu.sync_copy(idx_hbm.at[pl.ds(rbase, _CHUNK)], idx_v)
        pltpu.async_copy(table_hbm.at[idx_v], rows_v, sem).wait()
        pltpu.sync_copy(x_hbm.at[pl.ds(rbase, _CHUNK)], x_v)

        def body(r, acc):
            for c in range(_DIM // 16):
                q = rows_v[r, pl.ds(c * 16, 16)]
                xv = x_v[r, pl.ds(c * 16, 16)]
                dv = q - xv
                rows_v[r, pl.ds(c * 16, 16)] = xv + dv
                acc = acc + dv * dv
            return acc

        acc = lax.fori_loop(0, _CHUNK, body, acc)
        pltpu.sync_copy(rows_v, qst_hbm.at[pl.ds(rbase, _CHUNK)])
    acc_v[...] = acc
    pltpu.sync_copy(acc_v, loss_hbm.at[wid])


def _sc_gather(embed_t, ind_flat, flatten):
    mesh = plsc.VectorSubcoreMesh(core_axis_name="c", subcore_axis_name="s")
    fn = functools.partial(
        pl.kernel,
        mesh=mesh,
        out_type=[
            jax.ShapeDtypeStruct((_NTOK, _DIM), jnp.float32),
            jax.ShapeDtypeStruct((_NW, 16), jnp.float32),
        ],
        scratch_types=[
            pltpu.VMEM((_CHUNK,), jnp.int32),
            pltpu.VMEM((_CHUNK, _DIM), jnp.float32),
            pltpu.VMEM((_CHUNK, _DIM), jnp.float32),
            pltpu.VMEM((16,), jnp.float32),
            pltpu.SemaphoreType.DMA,
        ],
    )(_sc_gather_body)
    return fn(embed_t, ind_flat, flatten)


def kernel(input, embed):
    flatten = input.reshape(_NTOK, _DIM)
    ind_flat = _dist_argmin(flatten, embed)
    qst_flat, loss_partials = _sc_gather(embed.T, ind_flat, flatten)
    quantize_st = qst_flat.reshape(input.shape)
    embed_ind = ind_flat.reshape(input.shape[:-1])
    commit_loss = jnp.sum(loss_partials) / jnp.float32(_NTOK * _DIM)
    return quantize_st, embed_ind, commit_loss
